# baseline (speedup 1.0000x reference)
import functools

import jax
import jax.numpy as jnp
from jax import lax
from jax.experimental import pallas as pl
from jax.experimental.pallas import tpu as pltpu
from jax.experimental.pallas import tpu_sc as plsc

N, L, D = 1600, 77, 768
N_CTX = 8
CTX_POS = 1
NC, NS = 2, 16
NW = NC * NS
PPW = N // NW
NBUF = 5
NCHUNK = PPW // NBUF
MW = 25
MROWS = N // MW
LANES = 16
CHD = D // LANES
TAIL = L - 9 * N_CTX

_mesh = plsc.VectorSubcoreMesh(core_axis_name="c", subcore_axis_name="s")


@functools.partial(
    pl.kernel,
    mesh=_mesh,
    out_type=[
        jax.ShapeDtypeStruct((N, L, D), jnp.float32),
        jax.ShapeDtypeStruct((N, L), jnp.int32),
    ],
    scratch_types=[
        pltpu.VMEM((N_CTX, D), jnp.float32),
        pltpu.VMEM((NBUF, N_CTX, D), jnp.float32),
        pltpu.VMEM((NBUF, N_CTX, D), jnp.float32),
        pltpu.VMEM((NBUF, N_CTX, D), jnp.float32),
        pltpu.VMEM((MROWS, L), jnp.int32),
        pltpu.VMEM((MROWS, L), jnp.int32),
        pltpu.SemaphoreType.DMA,
        pltpu.SemaphoreType.DMA,
        pltpu.SemaphoreType.DMA,
        pltpu.SemaphoreType.DMA,
    ],
)
def _assemble(emb, ctx, msk, out_emb, out_msk,
              ctx_v, h1_v, b2_v, t_v, mi_v, mo_v,
              semA, semG, semS, semM):
    wid = lax.axis_index("s") * NC + lax.axis_index("c")
    base = wid * PPW

    bulk = pltpu.make_async_copy(
        emb.at[pl.ds(base, PPW), pl.ds(N_CTX, 7 * N_CTX)],
        out_emb.at[pl.ds(base, PPW), pl.ds(2 * N_CTX, 7 * N_CTX)],
        semA,
    )
    bulk.start()

    pltpu.sync_copy(ctx, ctx_v)

    def fill_h1(k, carry):
        for b in range(NBUF):
            for j in range(N_CTX - 1):
                h1_v[b, 1 + j, pl.ds(k * LANES, LANES)] = \
                    ctx_v[j, pl.ds(k * LANES, LANES)]
        return carry

    lax.fori_loop(0, CHD, fill_h1, 0)

    mask_in = pltpu.make_async_copy(
        msk.at[pl.ds(wid * MROWS, MROWS)], mi_v, semM)
    mask_out = pltpu.make_async_copy(
        mo_v, out_msk.at[pl.ds(wid * MROWS, MROWS)], semM)

    @pl.when(wid < 0)
    def _mask():
        mask_in.start()
        mask_in.wait()
        ones16 = jnp.full((LANES,), 1, jnp.int32)

        def mrow(i, carry):
            mo_v[i, pl.ds(0, LANES)] = mi_v[i, pl.ds(0, LANES)]
            mo_v[i, pl.ds(CTX_POS, LANES)] = ones16
            for k in range(4):
                mo_v[i, pl.ds(9 + 16 * k, LANES)] = \
                    mi_v[i, pl.ds(1 + 16 * k, LANES)]
            mo_v[i, pl.ds(L - LANES, LANES)] = \
                mi_v[i, pl.ds(L - LANES - N_CTX, LANES)]
            return carry

        lax.fori_loop(0, MROWS, mrow, 0)
        mask_out.start()

    def chunk(c, carry):
        p0 = base + c * NBUF
        gathers, scatters = [], []
        for b in range(NBUF):
            g1 = pltpu.make_async_copy(
                emb.at[p0 + b, pl.ds(0, N_CTX)], b2_v.at[b], semG)
            g2 = pltpu.make_async_copy(
                emb.at[p0 + b, pl.ds(8 * N_CTX, N_CTX)], t_v.at[b], semG)
            g1.start()
            g2.start()
            gathers += [g1, g2]
        for g in gathers:
            g.wait()
        for b in range(NBUF):
            def row0(k, carry2):
                h1_v[b, 0, pl.ds(k * LANES, LANES)] = \
                    b2_v[b, 0, pl.ds(k * LANES, LANES)]
                b2_v[b, 0, pl.ds(k * LANES, LANES)] = \
                    ctx_v[N_CTX - 1, pl.ds(k * LANES, LANES)]
                return carry2

            lax.fori_loop(0, CHD, row0, 0)
            s1 = pltpu.make_async_copy(
                h1_v.at[b], out_emb.at[p0 + b, pl.ds(0, N_CTX)], semS)
            s2 = pltpu.make_async_copy(
                b2_v.at[b], out_emb.at[p0 + b, pl.ds(N_CTX, N_CTX)], semS)
            s3 = pltpu.make_async_copy(
                t_v.at[b, pl.ds(0, TAIL)],
                out_emb.at[p0 + b, pl.ds(9 * N_CTX, TAIL)], semS)
            s1.start()
            s2.start()
            s3.start()
            scatters += [s1, s2, s3]
        for s in scatters:
            s.wait()
        return carry

    if False:
        lax.fori_loop(0, NCHUNK, chunk, 0)

    bulk.wait()

    @pl.when(wid < 0)
    def _mask_drain():
        mask_out.wait()


def kernel(token_emb_fixed, ctx, attn_mask, positional_embedding):
    del positional_embedding
    return tuple(_assemble(token_emb_fixed, ctx, attn_mask))


# speedup vs baseline: 10.8075x; 10.8075x over previous
import functools

import jax
import jax.numpy as jnp
from jax import lax
from jax.experimental import pallas as pl
from jax.experimental.pallas import tpu as pltpu
from jax.experimental.pallas import tpu_sc as plsc

N, L, D = 1600, 77, 768
N_CTX = 8
CTX_POS = 1
NC, NS = 2, 16
NW = NC * NS
PPW = N // NW
MBR = 48
MER = 8
LANES = 16
CHD = D // LANES
MID = 7 * N_CTX
TAIL = L - 9 * N_CTX

_mesh = plsc.VectorSubcoreMesh(core_axis_name="c", subcore_axis_name="s")


@functools.partial(
    pl.kernel,
    mesh=_mesh,
    out_type=[
        jax.ShapeDtypeStruct((N, L, D), jnp.float32),
        jax.ShapeDtypeStruct((N, L), jnp.int32),
    ],
    scratch_types=[
        pltpu.VMEM((2, MID, D), jnp.float32),
        pltpu.VMEM((2, N_CTX, D), jnp.float32),
        pltpu.VMEM((2, N_CTX, D), jnp.float32),
        pltpu.VMEM((2, N_CTX, D), jnp.float32),
        pltpu.VMEM((1, D), jnp.float32),
        pltpu.VMEM((MBR, L), jnp.int32),
        pltpu.SemaphoreType.DMA,
        pltpu.SemaphoreType.DMA,
        pltpu.SemaphoreType.DMA,
        pltpu.SemaphoreType.DMA,
        pltpu.SemaphoreType.DMA,
        pltpu.SemaphoreType.DMA,
        pltpu.SemaphoreType.DMA,
    ],
)
def _assemble(emb, ctx, msk, out_emb, out_msk,
              g_v, t_v, w1_v, w2_v, c7_v, m_v,
              semG0, semG1, semW0, semW1, semS0, semS1, semM):
    wid = lax.axis_index("s") * NC + lax.axis_index("c")
    base = wid * PPW
    semG = (semG0, semG1)
    semW = (semW0, semW1)
    semS = (semS0, semS1)

    ones16 = jnp.full((LANES,), 1, jnp.int32)

    def mrow(i, carry):
        a = [m_v[i, pl.ds(1 + 16 * k, LANES)] for k in range(4)]
        a.append(m_v[i, pl.ds(L - LANES - N_CTX, LANES)])
        m_v[i, pl.ds(CTX_POS, LANES)] = ones16
        for k in range(4):
            m_v[i, pl.ds(9 + 16 * k, LANES)] = a[k]
        m_v[i, pl.ds(L - LANES, LANES)] = a[4]
        return carry

    def mask_pass(row0, nrows):
        gin = pltpu.make_async_copy(
            msk.at[pl.ds(row0, nrows)], m_v.at[pl.ds(0, nrows)], semM)
        gin.start()
        gin.wait()
        lax.fori_loop(0, nrows, mrow, 0)
        return pltpu.make_async_copy(
            m_v.at[pl.ds(0, nrows)], out_msk.at[pl.ds(row0, nrows)], semM)

    mask_out1 = mask_pass(wid * MBR, MBR)
    mask_out1.start()

    pltpu.sync_copy(ctx, w1_v.at[0])

    def init_chunk(k, carry):
        c7_v[0, pl.ds(k * LANES, LANES)] = w1_v[0, 7, pl.ds(k * LANES, LANES)]
        for j in range(6, -1, -1):
            w1_v[0, j + 1, pl.ds(k * LANES, LANES)] = \
                w1_v[0, j, pl.ds(k * LANES, LANES)]
        for j in range(1, N_CTX):
            w1_v[1, j, pl.ds(k * LANES, LANES)] = \
                w1_v[0, j, pl.ds(k * LANES, LANES)]
        return carry

    lax.fori_loop(0, CHD, init_chunk, 0)

    def gathers(s, n):
        return (
            pltpu.make_async_copy(
                emb.at[n, pl.ds(N_CTX, MID)], g_v.at[s], semG[s]),
            pltpu.make_async_copy(
                emb.at[n, pl.ds(N_CTX + MID, N_CTX)], t_v.at[s], semG[s]),
            pltpu.make_async_copy(
                emb.at[n, pl.ds(0, N_CTX)], w2_v.at[s], semW[s]),
        )

    def scatters(s, n):
        return (
            pltpu.make_async_copy(
                w1_v.at[s], out_emb.at[n, pl.ds(0, N_CTX)], semS[s]),
            pltpu.make_async_copy(
                w2_v.at[s], out_emb.at[n, pl.ds(N_CTX, N_CTX)], semS[s]),
            pltpu.make_async_copy(
                g_v.at[s], out_emb.at[n, pl.ds(2 * N_CTX, MID)], semS[s]),
            pltpu.make_async_copy(
                t_v.at[s, pl.ds(0, TAIL)],
                out_emb.at[n, pl.ds(9 * N_CTX, TAIL)], semS[s]),
        )

    def step(i, carry):
        n = base + i
        for s in range(2):
            @pl.when(i % 2 == s)
            def _slot(s=s):
                @pl.when(i >= 2)
                def _drain():
                    for cp in scatters(s, n):
                        cp.wait()

                gG, gT, gW = gathers(s, n)
                gG.start()
                gT.start()
                gW.start()
                gW.wait()

                def row0(k, carry2):
                    w1_v[s, 0, pl.ds(k * LANES, LANES)] = \
                        w2_v[s, 0, pl.ds(k * LANES, LANES)]
                    w2_v[s, 0, pl.ds(k * LANES, LANES)] = \
                        c7_v[0, pl.ds(k * LANES, LANES)]
                    return carry2

                lax.fori_loop(0, CHD, row0, 0)
                gG.wait()
                gT.wait()
                for cp in scatters(s, n):
                    cp.start()
        return carry

    lax.fori_loop(0, PPW, step, 0)
    for s in range(2):
        for cp in scatters(s, base):
            cp.wait()

    mask_out1.wait()

    @pl.when(wid < (N - NW * MBR) // MER)
    def _mask_pass2():
        out2 = mask_pass(NW * MBR + wid * MER, MER)
        out2.start()
        out2.wait()


def kernel(token_emb_fixed, ctx, attn_mask, positional_embedding):
    del positional_embedding
    return tuple(_assemble(token_emb_fixed, ctx, attn_mask))
